# Initial kernel scaffold; baseline (speedup 1.0000x reference)
import jax
import jax.numpy as jnp
from jax.experimental import pallas as pl


def kernel(token_ids, A_weight, B_weight):
    raise NotImplementedError("write your pallas kernel here")


# speedup vs baseline: 12.0287x; 12.0287x over previous
import functools

import jax
import jax.numpy as jnp
from jax import lax
from jax.experimental import pallas as pl
from jax.experimental.pallas import tpu as pltpu
from jax.experimental.pallas import tpu_sc as plsc

VOCAB = 1000000
K = 32
EMBED_DIM = 64
B_TOK = 16384
L = 50
N = B_TOK * L

NUM_CORES = 2
NUM_SUBCORES = 16
NW = NUM_CORES * NUM_SUBCORES
PER_W = N // NW
CHUNK = 3200
NCHUNK = PER_W // CHUNK

_MESH = plsc.VectorSubcoreMesh(core_axis_name="c", subcore_axis_name="s")


@functools.partial(
    pl.kernel,
    mesh=_MESH,
    out_type=jax.ShapeDtypeStruct((N, K), jnp.float32),
    scratch_types=[
        pltpu.VMEM((CHUNK,), jnp.int32),
        pltpu.VMEM((CHUNK, K), jnp.float32),
        pltpu.SemaphoreType.DMA,
    ],
    compiler_params=pltpu.CompilerParams(use_tc_tiling_on_sc=False),
)
def _sc_gather(ids_hbm, table_hbm, out_hbm, idx_v, rows_v, sem):
    wid = lax.axis_index("s") * NUM_CORES + lax.axis_index("c")
    for g in range(NCHUNK):
        base = wid * PER_W + g * CHUNK
        pltpu.sync_copy(ids_hbm.at[pl.ds(base, CHUNK)], idx_v)
        pltpu.async_copy(table_hbm.at[idx_v], rows_v, sem).wait()
        pltpu.sync_copy(rows_v, out_hbm.at[pl.ds(base, CHUNK)])


_MM_BLK = 4096


def _mm_body(x_ref, w_ref, o_ref):
    o_ref[...] = lax.dot_general(
        x_ref[...], w_ref[...],
        dimension_numbers=(((1,), (1,)), ((), ())),
        preferred_element_type=jnp.float32,
    )


def _tc_project(low_dim, B_weight):
    return pl.pallas_call(
        _mm_body,
        grid=(N // _MM_BLK,),
        in_specs=[
            pl.BlockSpec((_MM_BLK, K), lambda i: (i, 0)),
            pl.BlockSpec((EMBED_DIM, K), lambda i: (0, 0)),
        ],
        out_specs=pl.BlockSpec((_MM_BLK, EMBED_DIM), lambda i: (i, 0)),
        out_shape=jax.ShapeDtypeStruct((N, EMBED_DIM), jnp.float32),
    )(low_dim, B_weight)


def kernel(token_ids, A_weight, B_weight):
    flat_ids = token_ids.reshape(-1).astype(jnp.int32)
    low_dim = _sc_gather(flat_ids, A_weight)
    full = _tc_project(low_dim, B_weight)
    return full.reshape(B_TOK, L, EMBED_DIM)


# speedup vs baseline: 22.6833x; 1.8858x over previous
import functools

import jax
import jax.numpy as jnp
from jax import lax
from jax.experimental import pallas as pl
from jax.experimental.pallas import tpu as pltpu
from jax.experimental.pallas import tpu_sc as plsc

VOCAB = 1000000
K = 32
EMBED_DIM = 64
B_TOK = 16384
L = 50
N = B_TOK * L


_CB = 4096
_P = 499712
_NPACK = _P + _CB


def _cbuild_body(at_lo_ref, at_hi_ref, b_ref, c_ref):
    bw = b_ref[...]
    lo = lax.dot_general(
        at_lo_ref[...], bw,
        dimension_numbers=(((0,), (1,)), ((), ())),
        preferred_element_type=jnp.float32,
    )
    hi = lax.dot_general(
        at_hi_ref[...], bw,
        dimension_numbers=(((0,), (1,)), ((), ())),
        preferred_element_type=jnp.float32,
    )
    c_ref[...] = lax.concatenate([lo, hi], 1)


def _tc_build_c(A_weight, B_weight):
    at = jnp.transpose(A_weight, (1, 0))
    return pl.pallas_call(
        _cbuild_body,
        grid=(_NPACK // _CB,),
        in_specs=[
            pl.BlockSpec((K, _CB), lambda i: (0, i)),
            pl.BlockSpec((K, _CB), lambda i: (0, i + _P // _CB)),
            pl.BlockSpec((EMBED_DIM, K), lambda i: (0, 0)),
        ],
        out_specs=pl.BlockSpec((_CB, 128), lambda i: (i, 0)),
        out_shape=jax.ShapeDtypeStruct((_NPACK, 128), jnp.float32),
    )(at, at, B_weight)


NUM_CORES = 2
NUM_SUBCORES = 16
NW = NUM_CORES * NUM_SUBCORES
PER_W = N // NW
CHUNK = 1600
NCHUNK = PER_W // CHUNK

_MESH = plsc.VectorSubcoreMesh(core_axis_name="c", subcore_axis_name="s")


@functools.partial(
    pl.kernel,
    mesh=_MESH,
    out_type=jax.ShapeDtypeStruct((N, EMBED_DIM), jnp.float32),
    name="sc_embed_gather",
    scratch_types=[
        pltpu.VMEM((CHUNK,), jnp.int32),
        pltpu.VMEM((CHUNK, EMBED_DIM), jnp.float32),
        pltpu.SemaphoreType.DMA,
    ],
    compiler_params=pltpu.CompilerParams(use_tc_tiling_on_sc=False),
)
def _sc_gather(ids_hbm, table_hbm, out_hbm, idx_v, rows_v, sem):
    wid = lax.axis_index("s") * NUM_CORES + lax.axis_index("c")
    for g in range(NCHUNK):
        base = wid * PER_W + g * CHUNK
        pltpu.sync_copy(ids_hbm.at[pl.ds(base, CHUNK)], idx_v)
        pltpu.async_copy(table_hbm.at[idx_v], rows_v, sem).wait()
        pltpu.sync_copy(rows_v, out_hbm.at[pl.ds(base, CHUNK)])


def kernel(token_ids, A_weight, B_weight):
    flat_ids = token_ids.reshape(-1).astype(jnp.int32)
    sub_ids = jnp.where(
        flat_ids < _P, 2 * flat_ids, 2 * (flat_ids - _P) + 1
    )
    c128 = _tc_build_c(A_weight, B_weight)
    c_table = c128.reshape(2 * _NPACK, EMBED_DIM)
    full = _sc_gather(sub_ids, c_table)
    return full.reshape(B_TOK, L, EMBED_DIM)


# speedup vs baseline: 36.8903x; 1.6263x over previous
import functools

import jax
import jax.numpy as jnp
from jax import lax
from jax.experimental import pallas as pl
from jax.experimental.pallas import tpu as pltpu
from jax.experimental.pallas import tpu_sc as plsc

VOCAB = 1000000
K = 32
EMBED_DIM = 64
B_TOK = 16384
HALF_B = B_TOK // 2
L = 50
N = B_TOK * L


_CB = 4096
_P = 499712
_NPACK = _P + _CB


def _cbuild_body(at_lo_ref, at_hi_ref, b_ref, c_ref):
    bw = b_ref[...]
    lo = lax.dot_general(
        at_lo_ref[...], bw,
        dimension_numbers=(((0,), (1,)), ((), ())),
        preferred_element_type=jnp.float32,
    )
    hi = lax.dot_general(
        at_hi_ref[...], bw,
        dimension_numbers=(((0,), (1,)), ((), ())),
        preferred_element_type=jnp.float32,
    )
    c_ref[...] = lax.concatenate([lo, hi], 1)


def _tc_build_c(A_weight, B_weight):
    at = jnp.transpose(A_weight, (1, 0))
    return pl.pallas_call(
        _cbuild_body,
        grid=(_NPACK // _CB,),
        in_specs=[
            pl.BlockSpec((K, _CB), lambda i: (0, i)),
            pl.BlockSpec((K, _CB), lambda i: (0, i + _P // _CB)),
            pl.BlockSpec((EMBED_DIM, K), lambda i: (0, 0)),
        ],
        out_specs=pl.BlockSpec((_CB, 128), lambda i: (i, 0)),
        out_shape=jax.ShapeDtypeStruct((_NPACK, 128), jnp.float32),
    )(at, at, B_weight)


NUM_CORES = 2
NUM_SUBCORES = 16
NW = NUM_CORES * NUM_SUBCORES
PER_W = N // NW
CHUNK = 1600
NCHUNK = PER_W // CHUNK

_MESH = plsc.VectorSubcoreMesh(core_axis_name="c", subcore_axis_name="s")


@functools.partial(
    pl.kernel,
    mesh=_MESH,
    out_type=jax.ShapeDtypeStruct((N, EMBED_DIM), jnp.float32),
    name="sc_embed_gather",
    scratch_types=[
        pltpu.VMEM((CHUNK,), jnp.int32),
        pltpu.VMEM((CHUNK,), jnp.int32),
        pltpu.VMEM((CHUNK, EMBED_DIM), jnp.float32),
        pltpu.SemaphoreType.DMA,
        pltpu.SemaphoreType.DMA,
    ],
    compiler_params=pltpu.CompilerParams(use_tc_tiling_on_sc=False),
)
def _sc_gather(ids_hbm, dest_hbm, table_hbm, out_hbm, idx_v, dst_v, rows_v,
               sem_g, sem_s):
    wid = lax.axis_index("s") * NUM_CORES + lax.axis_index("c")
    for g in range(NCHUNK):
        base = wid * PER_W + g * CHUNK
        pltpu.sync_copy(ids_hbm.at[pl.ds(base, CHUNK)], idx_v)
        pltpu.sync_copy(dest_hbm.at[pl.ds(base, CHUNK)], dst_v)
        pltpu.async_copy(table_hbm.at[idx_v], rows_v, sem_g).wait()
        pltpu.async_copy(rows_v, out_hbm.at[dst_v], sem_s).wait()


def _xpose_body(x_ref, z_ref):
    vt = lax.transpose(x_ref[0], (1, 0))
    z_ref[0, :, 0:HALF_B] = vt[0:EMBED_DIM, :]
    z_ref[0, :, HALF_B:B_TOK] = vt[EMBED_DIM:128, :]


def _tc_transpose(flat_perm):
    x3 = flat_perm.reshape(L, B_TOK // 2, 128)
    return pl.pallas_call(
        _xpose_body,
        grid=(L,),
        in_specs=[pl.BlockSpec((1, B_TOK // 2, 128), lambda i: (i, 0, 0))],
        out_specs=pl.BlockSpec((1, EMBED_DIM, B_TOK), lambda i: (i, 0, 0)),
        out_shape=jax.ShapeDtypeStruct((L, EMBED_DIM, B_TOK), jnp.float32),
    )(x3)


def kernel(token_ids, A_weight, B_weight):
    flat_ids = token_ids.reshape(-1).astype(jnp.int32)
    sub_ids = jnp.where(flat_ids < _P, 2 * flat_ids, 2 * (flat_ids - _P) + 1)
    n = jnp.arange(N, dtype=jnp.int32)
    b = n // L
    l = n % L
    dest_ids = l * B_TOK + jnp.where(b < HALF_B, 2 * b, 2 * (b - HALF_B) + 1)
    c128 = _tc_build_c(A_weight, B_weight)
    c_table = c128.reshape(2 * _NPACK, EMBED_DIM)
    flat_perm = _sc_gather(sub_ids, dest_ids, c_table)
    z = _tc_transpose(flat_perm)
    return jnp.transpose(z, (2, 0, 1))


# speedup vs baseline: 38.4837x; 1.0432x over previous
import functools

import jax
import jax.numpy as jnp
from jax import lax
from jax.experimental import pallas as pl
from jax.experimental.pallas import tpu as pltpu
from jax.experimental.pallas import tpu_sc as plsc

VOCAB = 1000000
K = 32
EMBED_DIM = 64
B_TOK = 16384
HALF_B = B_TOK // 2
L = 50
N = B_TOK * L


_CB = 8192
_P = 499712
_NPACK = _P + _CB


def _cbuild_body(at_lo_ref, at_hi_ref, b_ref, c_ref):
    bw = b_ref[...]
    lo = lax.dot_general(
        at_lo_ref[...], bw,
        dimension_numbers=(((0,), (1,)), ((), ())),
        preferred_element_type=jnp.float32,
    )
    hi = lax.dot_general(
        at_hi_ref[...], bw,
        dimension_numbers=(((0,), (1,)), ((), ())),
        preferred_element_type=jnp.float32,
    )
    c_ref[...] = lax.concatenate([lo, hi], 1)


def _tc_build_c(A_weight, B_weight):
    at = jnp.transpose(A_weight, (1, 0))
    return pl.pallas_call(
        _cbuild_body,
        grid=(_NPACK // _CB,),
        in_specs=[
            pl.BlockSpec((K, _CB), lambda i: (0, i)),
            pl.BlockSpec((K, _CB), lambda i: (0, i + _P // _CB)),
            pl.BlockSpec((EMBED_DIM, K), lambda i: (0, 0)),
        ],
        out_specs=pl.BlockSpec((_CB, 128), lambda i: (i, 0)),
        out_shape=jax.ShapeDtypeStruct((_NPACK, 128), jnp.float32),
    )(at, at, B_weight)


NUM_CORES = 2
NUM_SUBCORES = 16
NW = NUM_CORES * NUM_SUBCORES
PER_W = N // NW
CHUNK = 800
NCHUNK = PER_W // CHUNK

_MESH = plsc.VectorSubcoreMesh(core_axis_name="c", subcore_axis_name="s")


@functools.partial(
    pl.kernel,
    mesh=_MESH,
    out_type=jax.ShapeDtypeStruct((N, EMBED_DIM), jnp.float32),
    name="sc_embed_gather",
    scratch_types=[
        pltpu.VMEM((2, CHUNK), jnp.int32),
        pltpu.VMEM((2, CHUNK), jnp.int32),
        pltpu.VMEM((2, CHUNK, EMBED_DIM), jnp.float32),
        pltpu.SemaphoreType.DMA,
        pltpu.SemaphoreType.DMA,
        pltpu.SemaphoreType.DMA,
    ],
    compiler_params=pltpu.CompilerParams(use_tc_tiling_on_sc=False),
)
def _sc_gather(ids_hbm, dest_hbm, table_hbm, out_hbm, idx_v, dst_v, rows_v,
               sem_g, sem_s0, sem_s1):
    wid = lax.axis_index("s") * NUM_CORES + lax.axis_index("c")
    sem_s = (sem_s0, sem_s1)
    scatters = [None, None]
    for g in range(NCHUNK):
        b = g & 1
        if scatters[b] is not None:
            scatters[b].wait()
        base = wid * PER_W + g * CHUNK
        pltpu.sync_copy(ids_hbm.at[pl.ds(base, CHUNK)], idx_v.at[b])
        pltpu.sync_copy(dest_hbm.at[pl.ds(base, CHUNK)], dst_v.at[b])
        pltpu.async_copy(table_hbm.at[idx_v.at[b]], rows_v.at[b], sem_g).wait()
        sc = pltpu.make_async_copy(rows_v.at[b], out_hbm.at[dst_v.at[b]], sem_s[b])
        sc.start()
        scatters[b] = sc
    scatters[0].wait()
    scatters[1].wait()


def _xpose_body(x_ref, z_ref):
    vt = lax.transpose(x_ref[0], (1, 0))
    z_ref[0, :, 0:HALF_B] = vt[0:EMBED_DIM, :]
    z_ref[0, :, HALF_B:B_TOK] = vt[EMBED_DIM:128, :]


def _tc_transpose(flat_perm):
    x3 = flat_perm.reshape(L, B_TOK // 2, 128)
    return pl.pallas_call(
        _xpose_body,
        grid=(L,),
        in_specs=[pl.BlockSpec((1, B_TOK // 2, 128), lambda i: (i, 0, 0))],
        out_specs=pl.BlockSpec((1, EMBED_DIM, B_TOK), lambda i: (i, 0, 0)),
        out_shape=jax.ShapeDtypeStruct((L, EMBED_DIM, B_TOK), jnp.float32),
    )(x3)


def kernel(token_ids, A_weight, B_weight):
    flat_ids = token_ids.reshape(-1).astype(jnp.int32)
    sub_ids = jnp.where(flat_ids < _P, 2 * flat_ids, 2 * (flat_ids - _P) + 1)
    n = jnp.arange(N, dtype=jnp.int32)
    b = n // L
    l = n % L
    dest_ids = l * B_TOK + jnp.where(b < HALF_B, 2 * b, 2 * (b - HALF_B) + 1)
    c128 = _tc_build_c(A_weight, B_weight)
    c_table = c128.reshape(2 * _NPACK, EMBED_DIM)
    flat_perm = _sc_gather(sub_ids, dest_ids, c_table)
    z = _tc_transpose(flat_perm)
    return jnp.transpose(z, (2, 0, 1))


# speedup vs baseline: 39.4816x; 1.0259x over previous
import functools

import jax
import jax.numpy as jnp
from jax import lax
from jax.experimental import pallas as pl
from jax.experimental.pallas import tpu as pltpu
from jax.experimental.pallas import tpu_sc as plsc

VOCAB = 1000000
K = 32
EMBED_DIM = 64
B_TOK = 16384
HALF_B = B_TOK // 2
L = 50
N = B_TOK * L


_CB = 8192
_P = 499712
_NPACK = _P + _CB


def _cbuild_body(at_lo_ref, at_hi_ref, b_ref, c_ref):
    bw = b_ref[...]
    lo = lax.dot_general(
        at_lo_ref[...], bw,
        dimension_numbers=(((0,), (1,)), ((), ())),
        preferred_element_type=jnp.float32,
    )
    hi = lax.dot_general(
        at_hi_ref[...], bw,
        dimension_numbers=(((0,), (1,)), ((), ())),
        preferred_element_type=jnp.float32,
    )
    c_ref[...] = lax.concatenate([lo, hi], 1)


def _tc_build_c(A_weight, B_weight):
    at = jnp.transpose(A_weight, (1, 0))
    return pl.pallas_call(
        _cbuild_body,
        grid=(_NPACK // _CB,),
        in_specs=[
            pl.BlockSpec((K, _CB), lambda i: (0, i)),
            pl.BlockSpec((K, _CB), lambda i: (0, i + _P // _CB)),
            pl.BlockSpec((EMBED_DIM, K), lambda i: (0, 0)),
        ],
        out_specs=pl.BlockSpec((_CB, 128), lambda i: (i, 0)),
        out_shape=jax.ShapeDtypeStruct((_NPACK, 128), jnp.float32),
    )(at, at, B_weight)


NUM_CORES = 2
NUM_SUBCORES = 16
NW = NUM_CORES * NUM_SUBCORES
PER_W = N // NW
CHUNK = 800
NCHUNK = PER_W // CHUNK

_MESH = plsc.VectorSubcoreMesh(core_axis_name="c", subcore_axis_name="s")


@functools.partial(
    pl.kernel,
    mesh=_MESH,
    out_type=jax.ShapeDtypeStruct((N, EMBED_DIM), jnp.float32),
    name="sc_embed_gather",
    scratch_types=[
        pltpu.VMEM((2, CHUNK), jnp.int32),
        pltpu.VMEM((2, CHUNK), jnp.int32),
        pltpu.VMEM((2, CHUNK, EMBED_DIM), jnp.float32),
        pltpu.SemaphoreType.DMA,
        pltpu.SemaphoreType.DMA,
        pltpu.SemaphoreType.DMA,
    ],
    compiler_params=pltpu.CompilerParams(use_tc_tiling_on_sc=False),
)
def _sc_gather(ids_hbm, dest_hbm, table_hbm, out_hbm, idx_v, dst_v, rows_v,
               sem_g, sem_s0, sem_s1):
    wid = lax.axis_index("s") * NUM_CORES + lax.axis_index("c")
    sem_s = (sem_s0, sem_s1)
    scatters = [None, None]
    for g in range(NCHUNK):
        b = g & 1
        if scatters[b] is not None:
            scatters[b].wait()
        base = wid * PER_W + g * CHUNK
        pltpu.sync_copy(ids_hbm.at[pl.ds(base, CHUNK)], idx_v.at[b])
        pltpu.sync_copy(dest_hbm.at[pl.ds(base, CHUNK)], dst_v.at[b])
        pltpu.async_copy(table_hbm.at[idx_v.at[b]], rows_v.at[b], sem_g).wait()
        sc = pltpu.make_async_copy(rows_v.at[b], out_hbm.at[dst_v.at[b]], sem_s[b])
        sc.start()
        scatters[b] = sc
    scatters[0].wait()
    scatters[1].wait()


def _xpose_body(x_ref, z_ref):
    vt = lax.transpose(x_ref[0], (1, 0))
    z_ref[0, :, 0:HALF_B] = vt[0:EMBED_DIM, :]
    z_ref[0, :, HALF_B:B_TOK] = vt[EMBED_DIM:128, :]


def _tc_transpose(flat_perm):
    x3 = flat_perm.reshape(L, B_TOK // 2, 128)
    return pl.pallas_call(
        _xpose_body,
        grid=(L,),
        in_specs=[pl.BlockSpec((1, B_TOK // 2, 128), lambda i: (i, 0, 0))],
        out_specs=pl.BlockSpec((1, EMBED_DIM, B_TOK), lambda i: (i, 0, 0)),
        out_shape=jax.ShapeDtypeStruct((L, EMBED_DIM, B_TOK), jnp.float32),
    )(x3)


def kernel(token_ids, A_weight, B_weight):
    ids_l = jnp.transpose(token_ids, (1, 0)).reshape(-1).astype(jnp.int32)
    sub_ids = jnp.where(ids_l < _P, 2 * ids_l, 2 * (ids_l - _P) + 1)
    j = jnp.arange(N, dtype=jnp.int32)
    b = j & (B_TOK - 1)
    dest_ids = (j - b) + ((2 * b) & (B_TOK - 1)) + (b >> 13)
    c128 = _tc_build_c(A_weight, B_weight)
    c_table = c128.reshape(2 * _NPACK, EMBED_DIM)
    flat_perm = _sc_gather(sub_ids, dest_ids, c_table)
    z = _tc_transpose(flat_perm)
    return jnp.transpose(z, (2, 0, 1))


# speedup vs baseline: 45.7777x; 1.1595x over previous
import functools

import jax
import jax.numpy as jnp
from jax import lax
from jax.experimental import pallas as pl
from jax.experimental.pallas import tpu as pltpu
from jax.experimental.pallas import tpu_sc as plsc

VOCAB = 1000000
K = 32
EMBED_DIM = 64
B_TOK = 16384
HALF_B = B_TOK // 2
L = 50
N = B_TOK * L


_CB = 8192
_P = 499712
_NPACK = _P + _CB


def _cbuild_body(at_lo_ref, at_hi_ref, w2_ref, c_ref):
    at2 = lax.concatenate([at_lo_ref[...], at_hi_ref[...]], 0)
    c_ref[...] = lax.dot_general(
        at2, w2_ref[...],
        dimension_numbers=(((0,), (0,)), ((), ())),
        preferred_element_type=jnp.float32,
    )


def _tc_build_c(A_weight, B_weight):
    at = jnp.transpose(A_weight, (1, 0))
    bt = B_weight.T
    w2 = jnp.zeros((2 * K, 128), jnp.float32)
    w2 = w2.at[0:K, 0:EMBED_DIM].set(bt).at[K:2 * K, EMBED_DIM:128].set(bt)
    return pl.pallas_call(
        _cbuild_body,
        grid=(_NPACK // _CB,),
        in_specs=[
            pl.BlockSpec((K, _CB), lambda i: (0, i)),
            pl.BlockSpec((K, _CB), lambda i: (0, i + _P // _CB)),
            pl.BlockSpec((2 * K, 128), lambda i: (0, 0)),
        ],
        out_specs=pl.BlockSpec((_CB, 128), lambda i: (i, 0)),
        out_shape=jax.ShapeDtypeStruct((_NPACK, 128), jnp.float32),
    )(at, at, w2)


NUM_CORES = 2
NUM_SUBCORES = 16
NW = NUM_CORES * NUM_SUBCORES
PER_W = N // NW
CHUNK = 800
NCHUNK = PER_W // CHUNK

_MESH = plsc.VectorSubcoreMesh(core_axis_name="c", subcore_axis_name="s")


@functools.partial(
    pl.kernel,
    mesh=_MESH,
    out_type=jax.ShapeDtypeStruct((N, EMBED_DIM), jnp.float32),
    name="sc_embed_gather",
    scratch_types=[
        pltpu.VMEM((2, CHUNK), jnp.int32),
        pltpu.VMEM((2, CHUNK), jnp.int32),
        pltpu.VMEM((2, CHUNK, EMBED_DIM), jnp.float32),
        pltpu.SemaphoreType.DMA,
        pltpu.SemaphoreType.DMA,
        pltpu.SemaphoreType.DMA,
    ],
    compiler_params=pltpu.CompilerParams(use_tc_tiling_on_sc=False),
)
def _sc_gather(ids_hbm, dest_hbm, table_hbm, out_hbm, idx_v, dst_v, rows_v,
               sem_g, sem_s0, sem_s1):
    wid = lax.axis_index("s") * NUM_CORES + lax.axis_index("c")
    sem_s = (sem_s0, sem_s1)
    scatters = [None, None]
    for g in range(NCHUNK):
        b = g & 1
        if scatters[b] is not None:
            scatters[b].wait()
        base = wid * PER_W + g * CHUNK
        pltpu.sync_copy(ids_hbm.at[pl.ds(base, CHUNK)], idx_v.at[b])
        pltpu.sync_copy(dest_hbm.at[pl.ds(base, CHUNK)], dst_v.at[b])
        pltpu.async_copy(table_hbm.at[idx_v.at[b]], rows_v.at[b], sem_g).wait()
        sc = pltpu.make_async_copy(rows_v.at[b], out_hbm.at[dst_v.at[b]], sem_s[b])
        sc.start()
        scatters[b] = sc
    scatters[0].wait()
    scatters[1].wait()


def _xpose_body(x_ref, z_ref):
    vt = lax.transpose(x_ref[0], (1, 0))
    z_ref[0, :, 0:HALF_B] = vt[0:EMBED_DIM, :]
    z_ref[0, :, HALF_B:B_TOK] = vt[EMBED_DIM:128, :]


def _tc_transpose(flat_perm):
    x3 = flat_perm.reshape(L, B_TOK // 2, 128)
    return pl.pallas_call(
        _xpose_body,
        grid=(L,),
        in_specs=[pl.BlockSpec((1, B_TOK // 2, 128), lambda i: (i, 0, 0))],
        out_specs=pl.BlockSpec((1, EMBED_DIM, B_TOK), lambda i: (i, 0, 0)),
        out_shape=jax.ShapeDtypeStruct((L, EMBED_DIM, B_TOK), jnp.float32),
    )(x3)


def kernel(token_ids, A_weight, B_weight):
    ids_l = jnp.transpose(token_ids, (1, 0)).reshape(-1).astype(jnp.int32)
    sub_ids = jnp.where(ids_l < _P, 2 * ids_l, 2 * (ids_l - _P) + 1)
    j = jnp.arange(N, dtype=jnp.int32)
    b = j & (B_TOK - 1)
    dest_ids = (j - b) + ((2 * b) & (B_TOK - 1)) + (b >> 13)
    c128 = _tc_build_c(A_weight, B_weight)
    c_table = c128.reshape(2 * _NPACK, EMBED_DIM)
    flat_perm = _sc_gather(sub_ids, dest_ids, c_table)
    z = _tc_transpose(flat_perm)
    return jnp.transpose(z, (2, 0, 1))
